# baseline (speedup 1.0000x reference)
import functools

import jax
import jax.numpy as jnp
from jax import lax
from jax.experimental import pallas as pl
from jax.experimental.pallas import tpu as pltpu
from jax.experimental.pallas import tpu_sc as plsc

_NUM_EMB = 100000
_D = 64
_S = 12
_SP = 16
_B = 1024
_L = 50
_TOKENS = _B * _L

_NC = 2
_NS = 16
_NW = _NC * _NS
_TPW = _TOKENS // _NW

_C = 32
_ROWS = _C * _S
_NIDX = _ROWS // 128
_CHUNKS = _TPW // _C

_BLK = 4096
_NB = 25
_TAB_DATA = _NB * _BLK
_ZROWS = 8192
_TAB_ROWS = _TAB_DATA + _ZROWS


def _prep_body(w_ref, wr_ref, o_ref, wr16_ref):
    i = pl.program_id(0)
    x = w_ref[...]
    xa = x[:, :_D]
    xb = x[:, _D:]
    ssa = jnp.sum(xa * xa, axis=-1, keepdims=True)
    ssb = jnp.sum(xb * xb, axis=-1, keepdims=True)
    sa = jnp.where(ssa > 1.0, lax.rsqrt(ssa), 1.0)
    sb = jnp.where(ssb > 1.0, lax.rsqrt(ssb), 1.0)
    y = jnp.concatenate([xa * sa, xb * sb], axis=1)
    o_ref[...] = jnp.where(i < _NB, y, 0.0)
    w16 = jnp.pad(wr_ref[...], ((0, 0), (0, _SP - _S)))
    wr16_ref[...] = jnp.concatenate(
        [w16[k * (_BLK // 8):(k + 1) * (_BLK // 8)] for k in range(8)], axis=1)


def _prep(weight, wr):
    w128 = weight.reshape(_NUM_EMB // 2, 2 * _D)
    return pl.pallas_call(
        _prep_body,
        grid=(_NB + _ZROWS // _BLK,),
        in_specs=[
            pl.BlockSpec((_BLK // 2, 2 * _D), lambda i: (jnp.minimum(i, _NB - 1), 0)),
            pl.BlockSpec((_BLK, _S), lambda i: (jnp.minimum(i, _NB - 1), 0)),
        ],
        out_specs=[
            pl.BlockSpec((_BLK // 2, 2 * _D), lambda i: (i, 0)),
            pl.BlockSpec((_BLK // 8, 8 * _SP), lambda i: (jnp.minimum(i, _NB - 1), 0)),
        ],
        out_shape=[
            jax.ShapeDtypeStruct((_TAB_ROWS // 2, 2 * _D), jnp.float32),
            jax.ShapeDtypeStruct((_TAB_DATA // 8, 8 * _SP), jnp.int32),
        ],
    )(w128, wr)


def _bag_body(ids_hbm, wr_hbm, tab_hbm, out_hbm,
              ids_v, sub_v, idx0, idx1, emb0, emb1, out_v,
              sem_s, sem_b0, sem_b1):
    wid = lax.axis_index("s") * _NC + lax.axis_index("c")
    tbase = wid * _TPW
    lanes = lax.iota(jnp.int32, 16)

    pltpu.sync_copy(ids_hbm.at[pl.ds(tbase, _TPW)], ids_v)

    @plsc.parallel_loop(0, _TPW // 16, unroll=4)
    def permute(i):
        v = ids_v[pl.ds(i * 16, 16)]
        s = jnp.bitwise_and(v, _BLK - 1)
        ids_v[pl.ds(i * 16, 16)] = (v - s + 8 * jnp.bitwise_and(s, 511)
                                    + jnp.right_shift(s, 9))

    for k in range(_TPW // 128):
        pltpu.async_copy(wr_hbm.at[ids_v.at[pl.ds(k * 128, 128)]],
                         sub_v.at[pl.ds(k * 128, 128)], sem_s)
    rem = _TPW % 128
    if rem:
        pltpu.async_copy(wr_hbm.at[ids_v.at[pl.ds(_TPW - rem, rem)]],
                         sub_v.at[pl.ds(_TPW - rem, rem)], sem_s)
    pltpu.make_async_copy(wr_hbm.at[pl.ds(0, _TPW)], sub_v, sem_s).wait()

    def build(c, idx_ref):
        @plsc.parallel_loop(0, _C, unroll=4)
        def body(t):
            row = sub_v[c * _C + t, :]
            pos = lanes * _C + t
            spread = _TAB_DATA + jnp.bitwise_and(wid * _ROWS + pos, _ZROWS - 1)
            plsc.store_scatter(idx_ref, [pos],
                               jnp.where(row == 0, spread, row),
                               mask=lanes < _S)

    def fire(idx_ref, emb_ref, sem):
        for k in range(_NIDX):
            pltpu.async_copy(tab_hbm.at[idx_ref.at[pl.ds(k * 128, 128)]],
                             emb_ref.at[pl.ds(k * 128, 128)], sem)

    def wait_emb(emb_ref, sem):
        pltpu.make_async_copy(tab_hbm.at[pl.ds(0, _ROWS)], emb_ref, sem).wait()

    def compute_out(c, emb_ref):
        @plsc.parallel_loop(0, _C, unroll=2)
        def tok(t):
            for q in range(_D // 16):
                sl = pl.ds(q * 16, 16)
                acc = emb_ref[t, sl]
                for j in range(1, _S):
                    acc = acc + emb_ref[j * _C + t, sl]
                out_v[t, sl] = acc
        pltpu.sync_copy(out_v, out_hbm.at[pl.ds(tbase + c * _C, _C)])

    build(0, idx0)
    fire(idx0, emb0, sem_b0)

    def group(g, carry):
        a = 2 * g + 1
        build(a, idx1)
        fire(idx1, emb1, sem_b1)
        wait_emb(emb0, sem_b0)
        compute_out(a - 1, emb0)
        build(a + 1, idx0)
        fire(idx0, emb0, sem_b0)
        wait_emb(emb1, sem_b1)
        compute_out(a, emb1)
        return carry

    lax.fori_loop(0, (_CHUNKS - 2) // 2, group, 0)

    last = _CHUNKS - 1
    build(last, idx1)
    fire(idx1, emb1, sem_b1)
    wait_emb(emb0, sem_b0)
    compute_out(last - 1, emb0)
    wait_emb(emb1, sem_b1)
    compute_out(last, emb1)


@functools.partial(
    pl.kernel,
    out_type=jax.ShapeDtypeStruct((_TOKENS, _D), jnp.float32),
    mesh=plsc.VectorSubcoreMesh(core_axis_name="c", subcore_axis_name="s"),
    compiler_params=pltpu.CompilerParams(
        needs_layout_passes=False, use_tc_tiling_on_sc=False),
    scratch_types=[
        pltpu.VMEM((_TPW,), jnp.int32),
        pltpu.VMEM((_TPW, _SP), jnp.int32),
        pltpu.VMEM((_ROWS,), jnp.int32),
        pltpu.VMEM((_ROWS,), jnp.int32),
        pltpu.VMEM((_ROWS, _D), jnp.float32),
        pltpu.VMEM((_ROWS, _D), jnp.float32),
        pltpu.VMEM((_C, _D), jnp.float32),
        pltpu.SemaphoreType.DMA,
        pltpu.SemaphoreType.DMA,
        pltpu.SemaphoreType.DMA,
    ],
)
def _bag_kernel(ids_hbm, wr_hbm, tab_hbm, out_hbm,
                ids_v, sub_v, idx0, idx1, emb0, emb1, out_v,
                sem_s, sem_b0, sem_b1):
    _bag_body(ids_hbm, wr_hbm, tab_hbm, out_hbm,
              ids_v, sub_v, idx0, idx1, emb0, emb1, out_v,
              sem_s, sem_b0, sem_b1)


def kernel(input_ids, word_representation, weight):
    ids = input_ids.reshape(-1)
    tab128, wr16_128 = _prep(weight, word_representation)
    tab = tab128.reshape(_TAB_ROWS, _D)
    wr16 = wr16_128.reshape(_TAB_DATA, _SP)
    out = _bag_kernel(ids, wr16, tab)
    return out.reshape(_B, _L, _D)


# speedup vs baseline: 1.0628x; 1.0628x over previous
import functools

import jax
import jax.numpy as jnp
from jax import lax
from jax.experimental import pallas as pl
from jax.experimental.pallas import tpu as pltpu
from jax.experimental.pallas import tpu_sc as plsc

_NUM_EMB = 100000
_D = 64
_S = 12
_SP = 16
_B = 1024
_L = 50
_TOKENS = _B * _L

_NC = 2
_NS = 16
_NW = _NC * _NS
_TPW = _TOKENS // _NW

_C = 32
_ROWS = _C * _S
_NIDX = _ROWS // 128
_CHUNKS = _TPW // _C

_BLK = 4096
_NB = 25
_TAB_DATA = _NB * _BLK
_ZROWS = 8192
_TAB_ROWS = _TAB_DATA + _ZROWS


def _prep_body(w_ref, wr_ref, o_ref, wr16_ref):
    i = pl.program_id(0)
    x = w_ref[...]
    ss = jnp.sum(x * x, axis=-1, keepdims=True)
    scale = jnp.where(ss > 1.0, lax.rsqrt(ss), 1.0)
    y = jnp.where(i < _NB, x * scale, 0.0)
    o_ref[...] = jnp.concatenate([y[: _BLK // 2], y[_BLK // 2:]], axis=1)
    w16 = jnp.pad(wr_ref[...], ((0, 0), (0, _SP - _S)))
    wr16_ref[...] = jnp.concatenate(
        [w16[k * (_BLK // 8):(k + 1) * (_BLK // 8)] for k in range(8)], axis=1)


def _prep(weight, wr):
    return pl.pallas_call(
        _prep_body,
        grid=(_NB + _ZROWS // _BLK,),
        in_specs=[
            pl.BlockSpec((_BLK, _D), lambda i: (jnp.minimum(i, _NB - 1), 0)),
            pl.BlockSpec((_BLK, _S), lambda i: (jnp.minimum(i, _NB - 1), 0)),
        ],
        out_specs=[
            pl.BlockSpec((_BLK // 2, 2 * _D), lambda i: (i, 0)),
            pl.BlockSpec((_BLK // 8, 8 * _SP), lambda i: (jnp.minimum(i, _NB - 1), 0)),
        ],
        out_shape=[
            jax.ShapeDtypeStruct((_TAB_ROWS // 2, 2 * _D), jnp.float32),
            jax.ShapeDtypeStruct((_TAB_DATA // 8, 8 * _SP), jnp.int32),
        ],
    )(weight, wr)


def _bag_body(ids_hbm, wr_hbm, tab_hbm, out_hbm,
              ids_v, sub_v, idx0, idx1, emb0, emb1, out_v,
              sem_s, sem_b0, sem_b1):
    wid = lax.axis_index("s") * _NC + lax.axis_index("c")
    tbase = wid * _TPW
    lanes = lax.iota(jnp.int32, 16)

    pltpu.sync_copy(ids_hbm.at[pl.ds(tbase, _TPW)], ids_v)

    @plsc.parallel_loop(0, _TPW // 16, unroll=4)
    def permute(i):
        v = ids_v[pl.ds(i * 16, 16)]
        s = jnp.bitwise_and(v, _BLK - 1)
        ids_v[pl.ds(i * 16, 16)] = (v - s + 8 * jnp.bitwise_and(s, 511)
                                    + jnp.right_shift(s, 9))

    for k in range(_TPW // 128):
        pltpu.async_copy(wr_hbm.at[ids_v.at[pl.ds(k * 128, 128)]],
                         sub_v.at[pl.ds(k * 128, 128)], sem_s)
    rem = _TPW % 128
    if rem:
        pltpu.async_copy(wr_hbm.at[ids_v.at[pl.ds(_TPW - rem, rem)]],
                         sub_v.at[pl.ds(_TPW - rem, rem)], sem_s)
    pltpu.make_async_copy(wr_hbm.at[pl.ds(0, _TPW)], sub_v, sem_s).wait()

    def build(c, idx_ref):
        @plsc.parallel_loop(0, _C, unroll=4)
        def body(t):
            row = sub_v[c * _C + t, :]
            pos = lanes * _C + t
            r = jnp.bitwise_and(row, _BLK - 1)
            tabrow = (row - r + 2 * jnp.bitwise_and(r, _BLK // 2 - 1)
                      + jnp.right_shift(r, 11))
            spread = _TAB_DATA + jnp.bitwise_and(wid * _ROWS + pos, _ZROWS - 1)
            plsc.store_scatter(idx_ref, [pos],
                               jnp.where(row == 0, spread, tabrow),
                               mask=lanes < _S)

    def fire(idx_ref, emb_ref, sem):
        for k in range(_NIDX):
            pltpu.async_copy(tab_hbm.at[idx_ref.at[pl.ds(k * 128, 128)]],
                             emb_ref.at[pl.ds(k * 128, 128)], sem)

    def wait_emb(emb_ref, sem):
        pltpu.make_async_copy(tab_hbm.at[pl.ds(0, _ROWS)], emb_ref, sem).wait()

    def compute_out(c, emb_ref):
        @plsc.parallel_loop(0, _C, unroll=2)
        def tok(t):
            for q in range(_D // 16):
                sl = pl.ds(q * 16, 16)
                acc = emb_ref[t, sl]
                for j in range(1, _S):
                    acc = acc + emb_ref[j * _C + t, sl]
                out_v[t, sl] = acc
        pltpu.sync_copy(out_v, out_hbm.at[pl.ds(tbase + c * _C, _C)])

    build(0, idx0)
    fire(idx0, emb0, sem_b0)

    def group(g, carry):
        a = 2 * g + 1
        build(a, idx1)
        fire(idx1, emb1, sem_b1)
        wait_emb(emb0, sem_b0)
        compute_out(a - 1, emb0)
        build(a + 1, idx0)
        fire(idx0, emb0, sem_b0)
        wait_emb(emb1, sem_b1)
        compute_out(a, emb1)
        return carry

    lax.fori_loop(0, (_CHUNKS - 2) // 2, group, 0)

    last = _CHUNKS - 1
    build(last, idx1)
    fire(idx1, emb1, sem_b1)
    wait_emb(emb0, sem_b0)
    compute_out(last - 1, emb0)
    wait_emb(emb1, sem_b1)
    compute_out(last, emb1)


@functools.partial(
    pl.kernel,
    out_type=jax.ShapeDtypeStruct((_TOKENS, _D), jnp.float32),
    mesh=plsc.VectorSubcoreMesh(core_axis_name="c", subcore_axis_name="s"),
    compiler_params=pltpu.CompilerParams(
        needs_layout_passes=False, use_tc_tiling_on_sc=False),
    scratch_types=[
        pltpu.VMEM((_TPW,), jnp.int32),
        pltpu.VMEM((_TPW, _SP), jnp.int32),
        pltpu.VMEM((_ROWS,), jnp.int32),
        pltpu.VMEM((_ROWS,), jnp.int32),
        pltpu.VMEM((_ROWS, _D), jnp.float32),
        pltpu.VMEM((_ROWS, _D), jnp.float32),
        pltpu.VMEM((_C, _D), jnp.float32),
        pltpu.SemaphoreType.DMA,
        pltpu.SemaphoreType.DMA,
        pltpu.SemaphoreType.DMA,
    ],
)
def _bag_kernel(ids_hbm, wr_hbm, tab_hbm, out_hbm,
                ids_v, sub_v, idx0, idx1, emb0, emb1, out_v,
                sem_s, sem_b0, sem_b1):
    _bag_body(ids_hbm, wr_hbm, tab_hbm, out_hbm,
              ids_v, sub_v, idx0, idx1, emb0, emb1, out_v,
              sem_s, sem_b0, sem_b1)


def kernel(input_ids, word_representation, weight):
    ids = input_ids.reshape(-1)
    tab128, wr16_128 = _prep(weight, word_representation)
    tab = tab128.reshape(_TAB_ROWS, _D)
    wr16 = wr16_128.reshape(_TAB_DATA, _SP)
    out = _bag_kernel(ids, wr16, tab)
    return out.reshape(_B, _L, _D)
